# baseline (speedup 1.0000x reference)
import functools

import jax
import jax.numpy as jnp
from jax import lax
from jax.experimental import pallas as pl
from jax.experimental.pallas import tpu as pltpu
from jax.experimental.pallas import tpu_sc as plsc

F32 = jnp.float32
EMB = 32


def _cdiv(a, b):
    return -(-a // b)


def _proj_t(xt, wbs, m_out, bm=3584):
    k, e = xt.shape
    assert m_out % bm == 0
    bm4 = bm // 4
    mo4 = m_out // 4
    grid = m_out // bm
    n_in_blocks = _cdiv(e, bm)
    nw = len(wbs)
    dn = (((0,), (0,)), ((), ()))

    def body(x_ref, *refs):
        w_refs = refs[:nw]
        b_refs = refs[nw:2 * nw]
        o_refs = refs[2 * nw:]
        xb = x_ref[...]
        for w_r, b_r, o_r in zip(w_refs, b_refs, o_refs):
            y = lax.dot_general(xb, w_r[...], dn,
                                preferred_element_type=F32,
                                precision=lax.Precision.HIGHEST) + b_r[...]
            o_r[...] = jnp.concatenate(
                [y[q * bm4:(q + 1) * bm4] for q in range(4)], axis=1)

    in_specs = [pl.BlockSpec((k, bm),
                             lambda i, _n=n_in_blocks: (0, jnp.minimum(i, _n - 1)))]
    in_specs += [pl.BlockSpec((k, EMB), lambda i: (0, 0)) for _ in wbs]
    in_specs += [pl.BlockSpec((1, EMB), lambda i: (0, 0)) for _ in wbs]
    out_specs = [pl.BlockSpec((bm4, 4 * EMB), lambda i: (i, 0)) for _ in wbs]
    out_shape = [jax.ShapeDtypeStruct((mo4, 4 * EMB), F32) for _ in wbs]
    ws = [w for w, _ in wbs]
    bs = [b.reshape(1, EMB) for _, b in wbs]
    outs = pl.pallas_call(
        body, grid=(grid,), in_specs=in_specs, out_specs=out_specs,
        out_shape=out_shape)(xt, *ws, *bs)
    return [o.reshape(m_out, EMB) for o in outs]


def _pack_perm(idx, bm=3584):
    e_pad = idx.shape[0]
    return idx.reshape(e_pad // bm, 4, bm // 4).transpose(0, 2, 1).reshape(-1)


def _bd(w):
    return jnp.kron(jnp.eye(4, dtype=F32), w)


def _proj(x, wbs, m_out=None, bm=2048):
    m, k = x.shape
    mo = m_out if m_out is not None else m
    assert m % 4 == 0 and mo % 4 == 0 and bm % 4 == 0
    bm4 = bm // 4
    m4 = m // 4
    mo4 = mo // 4
    n_in_blocks = _cdiv(m4, bm4)
    grid = _cdiv(mo4, bm4)
    nw = len(wbs)
    x4 = x.reshape(m4, 4 * k)

    def body(x_ref, *refs):
        w_refs = refs[:nw]
        b_refs = refs[nw:2 * nw]
        o_refs = refs[2 * nw:]
        xb = x_ref[...]
        for w_r, b_r, o_r in zip(w_refs, b_refs, o_refs):
            o_r[...] = jnp.dot(xb, w_r[...], preferred_element_type=F32,
                               precision=lax.Precision.HIGHEST) + b_r[...]

    in_specs = [pl.BlockSpec((bm4, 4 * k),
                             lambda i, _n=n_in_blocks: (jnp.minimum(i, _n - 1), 0))]
    in_specs += [pl.BlockSpec((4 * k, 4 * EMB), lambda i: (0, 0)) for _ in wbs]
    in_specs += [pl.BlockSpec((1, 4 * EMB), lambda i: (0, 0)) for _ in wbs]
    out_specs = [pl.BlockSpec((bm4, 4 * EMB), lambda i: (i, 0)) for _ in wbs]
    out_shape = [jax.ShapeDtypeStruct((mo4, 4 * EMB), F32) for _ in wbs]
    ws = [_bd(w) for w, _ in wbs]
    bs = [jnp.tile(b, 4).reshape(1, 4 * EMB) for _, b in wbs]
    outs = pl.pallas_call(
        body, grid=(grid,), in_specs=in_specs, out_specs=out_specs,
        out_shape=out_shape)(x4, *ws, *bs)
    return [o.reshape(mo, EMB) for o in outs]


def _edge_aggr(tab_t, tab_s, erows, idx_t, idx_s, idx_o, n_acc, n_acc_pad):
    e_pad = erows.shape[0]
    CH = 112
    NW = 32
    assert e_pad % (NW * CH * 4) == 0
    iters = e_pad // (NW * CH)
    per_w = iters * CH
    assert iters >= 8 and iters % 4 == 0
    zrows = n_acc_pad // 16
    assert zrows % 16 == 0
    zc = next(z for z in range(112, 7, -8) if zrows % z == 0)
    n_zb = zrows // zc
    dr = ((_cdiv(n_acc, 16) + 7) // 8) * 8
    dr_last = n_acc - 15 * dr
    assert dr_last > 0 and dr % 8 == 0 and dr_last % 8 == 0
    dc = ((_cdiv(n_acc, 16) + 127) // 128) * 128
    dc_last = n_acc - 15 * dc
    assert dc_last > 0 and dc % 128 == 0

    mesh = plsc.VectorSubcoreMesh(core_axis_name="c", subcore_axis_name="s")

    @functools.partial(
        pl.kernel, mesh=mesh,
        compiler_params=pltpu.CompilerParams(use_tc_tiling_on_sc=False),
        out_type=[jax.ShapeDtypeStruct((2, n_acc, EMB), F32),
                  jax.ShapeDtypeStruct((n_acc,), F32),
                  jax.ShapeDtypeStruct((n_acc,), F32)],
        scratch_types=(
            [pltpu.VMEM((CH,), jnp.int32)] * 8
            + [pltpu.VMEM((CH, EMB), F32)] * 6
            + [
                pltpu.VMEM((CH,), F32),
                pltpu.VMEM((zc, EMB), F32),
                pltpu.VMEM((zc,), F32),
                pltpu.VMEM_SHARED((n_acc_pad, EMB), F32),
                pltpu.VMEM_SHARED((n_acc_pad,), F32),
                pltpu.SemaphoreType.DMA,
                pltpu.SemaphoreType.DMA,
                pltpu.SemaphoreType.DMA,
                pltpu.SemaphoreType.DMA,
                pltpu.SemaphoreType.DMA,
            ]
        ),
    )
    def k(tab_t_h, tab_s_h, erows_h, idx_t_h, idx_s_h, idx_o_h,
          out_sum, out_cnt0, out_cnt1,
          it0, is0, it1, is1, io0, io1, io2, io3,
          rt0, rs0, re0, rt1, rs1, re1,
          ones_v, zrow_v, zcnt_v,
          sum_sp, cnt_sp, semi0, semi1, semg0, semg1, sems):
        cid = lax.axis_index("c")
        sid = lax.axis_index("s")
        w = cid * 16 + sid

        def zr_body(i, c):
            zrow_v[i, pl.ds(0, 16)] = jnp.zeros((16,), F32)
            zrow_v[i, pl.ds(16, 16)] = jnp.zeros((16,), F32)
            return c
        lax.fori_loop(0, zc, zr_body, 0)

        def zc_body(i, c):
            zcnt_v[pl.ds(i * 16, 16)] = jnp.zeros((16,), F32)
            return c
        lax.fori_loop(0, zc // 16, zc_body, 0)

        def on_body(i, c):
            ones_v[pl.ds(i * 16, 16)] = jnp.ones((16,), F32)
            return c
        lax.fori_loop(0, CH // 16, on_body, 0)

        for q in range(n_zb):
            pltpu.sync_copy(zrow_v,
                            sum_sp.at[pl.ds(sid * zrows + q * zc, zc)])
            pltpu.sync_copy(zcnt_v,
                            cnt_sp.at[pl.ds(sid * zrows + q * zc, zc)])
        plsc.subcore_barrier()

        idxg = ((it0, is0), (it1, is1))
        idxo = (io0, io1, io2, io3)
        datb = ((rt0, rs0, re0), (rt1, rs1, re1))
        semi = (semi0, semi1)
        semg = (semg0, semg1)

        def issue_idx(i, b, o):
            base = pl.multiple_of(w * per_w + i * CH, 8)
            pltpu.async_copy(idx_t_h.at[pl.ds(base, CH)], idxg[b][0], semi[b])
            pltpu.async_copy(idx_s_h.at[pl.ds(base, CH)], idxg[b][1], semi[b])
            pltpu.async_copy(idx_o_h.at[pl.ds(base, CH)], idxo[o], semi[b])

        def wait_idx(b, o):
            for dst in (idxg[b][0], idxg[b][1], idxo[o]):
                pltpu.make_async_copy(idx_t_h.at[pl.ds(0, CH)], dst,
                                      semi[b]).wait()

        def issue_gather(i, b):
            base = pl.multiple_of(w * per_w + i * CH, 8)
            pltpu.async_copy(erows_h.at[pl.ds(base, CH)], datb[b][2], semg[b])
            pltpu.async_copy(tab_t_h.at[idxg[b][0]], datb[b][0], semg[b])
            pltpu.async_copy(tab_s_h.at[idxg[b][1]], datb[b][1], semg[b])

        def wait_gather(b):
            for dst in datb[b]:
                pltpu.make_async_copy(erows_h.at[pl.ds(0, CH)], dst,
                                      semg[b]).wait()

        def drain_scatter():
            pltpu.make_async_copy(erows_h.at[pl.ds(0, CH)], re0, sems).wait()

        def half(i, b, o):
            rt, rs, re = datb[b]
            wait_gather(b)

            def cbody(blk, cc):
                r0 = blk * 16
                for rr in range(16):
                    r = r0 + rr
                    for col in (0, 16):
                        z = (rt[r, pl.ds(col, 16)] + rs[r, pl.ds(col, 16)]
                             + re[r, pl.ds(col, 16)])
                        re[r, pl.ds(col, 16)] = jnp.maximum(z, 0.0)
                return cc
            lax.fori_loop(0, CH // 16, cbody, 0)

            @pl.when(i >= 1)
            def _():
                drain_scatter()
            pltpu.async_copy(re, sum_sp.at[idxo[o]], sems, add=True)
            pltpu.sync_copy(ones_v, cnt_sp.at[idxo[o]], add=True)

            @pl.when(i + 2 < iters)
            def _():
                issue_idx(i + 2, b, (o + 2) % 4)

            @pl.when(i + 1 < iters)
            def _():
                wait_idx(b ^ 1, (o + 1) % 4)
                issue_gather(i + 1, b ^ 1)

        issue_idx(0, 0, 0)
        wait_idx(0, 0)
        issue_gather(0, 0)
        issue_idx(1, 1, 1)

        def it_body(q, c):
            for j in range(4):
                half(4 * q + j, j & 1, j)
            return c
        lax.fori_loop(0, iters // 4, it_body, 0)
        drain_scatter()

        plsc.subcore_barrier()

        @pl.when(sid < 15)
        def _():
            pltpu.sync_copy(sum_sp.at[pl.ds(sid * dr, dr)],
                            out_sum.at[cid, pl.ds(sid * dr, dr)])

        @pl.when(sid == 15)
        def _():
            pltpu.sync_copy(sum_sp.at[pl.ds(15 * dr, dr_last)],
                            out_sum.at[cid, pl.ds(15 * dr, dr_last)])

        for c, out_cnt in ((0, out_cnt0), (1, out_cnt1)):
            @pl.when(jnp.logical_and(cid == c, sid < 15))
            def _(out_cnt=out_cnt):
                pltpu.sync_copy(cnt_sp.at[pl.ds(sid * dc, dc)],
                                out_cnt.at[pl.ds(sid * dc, dc)])

            @pl.when(jnp.logical_and(cid == c, sid == 15))
            def _(out_cnt=out_cnt):
                pltpu.sync_copy(cnt_sp.at[pl.ds(15 * dc, dc_last)],
                                out_cnt.at[pl.ds(15 * dc, dc_last)])

    return k(tab_t, tab_s, erows, idx_t, idx_s, idx_o)


def _post_v(sum_g, cnt_g0, cnt_g1, sum_h, cnt_h0, cnt_h1, pxv,
            gv_w2, gv_b2, hv_w2, hv_b2, w1g, w1h, fv_w2, fv_b2, w1f):
    m = cnt_g0.shape[0]
    bm = 2048
    bm4 = bm // 4
    m4 = m // 4
    grid = _cdiv(m4, bm4)
    L = 4 * EMB

    def rep(cnt):
        return jnp.repeat(cnt.reshape(m4, 4), EMB, axis=1)

    def body(sg_r, cg_r, sh_r, ch_r, px_r, gw2, gb2, hw2, hb2,
             w1g_r, w1h_r, fw2, fb2, w1f_r, out_r):
        def aggr(s_r, c_r, w2, b2):
            sarr = s_r[...]
            s = sarr[0] + sarr[1]
            c = c_r[...]
            mvals = s / jnp.maximum(c, 1.0)
            gate = jnp.where(c > 0.0, 1.0, 0.0)
            return (jnp.dot(mvals, w2[...], preferred_element_type=F32,
                            precision=lax.Precision.HIGHEST)
                    + b2[...] * gate)
        ag = aggr(sg_r, cg_r, gw2, gb2)
        ah = aggr(sh_r, ch_r, hw2, hb2)
        hmid = jax.nn.relu(
            px_r[...]
            + jnp.dot(ag, w1g_r[...], preferred_element_type=F32,
                      precision=lax.Precision.HIGHEST)
            + jnp.dot(ah, w1h_r[...], preferred_element_type=F32,
                      precision=lax.Precision.HIGHEST))
        fv = jnp.dot(hmid, fw2[...], preferred_element_type=F32,
                     precision=lax.Precision.HIGHEST) + fb2[...]
        out_r[...] = jnp.dot(fv, w1f_r[...], preferred_element_type=F32,
                             precision=lax.Precision.HIGHEST)

    sblk = pl.BlockSpec((2, bm4, L), lambda i: (0, i, 0))
    cblk = pl.BlockSpec((bm4, L), lambda i: (i, 0))
    wblk = pl.BlockSpec((L, L), lambda i: (0, 0))
    bblk = pl.BlockSpec((1, L), lambda i: (0, 0))
    out = pl.pallas_call(
        body, grid=(grid,),
        in_specs=[sblk, cblk, sblk, cblk, cblk,
                  wblk, bblk, wblk, bblk, wblk, wblk, wblk, bblk, wblk],
        out_specs=pl.BlockSpec((bm4, L), lambda i: (i, 0)),
        out_shape=jax.ShapeDtypeStruct((m4, L), F32),
    )(sum_g.reshape(2, m4, L), rep(cnt_g0 + cnt_g1),
      sum_h.reshape(2, m4, L), rep(cnt_h0 + cnt_h1),
      pxv.reshape(m4, L),
      _bd(gv_w2), jnp.tile(gv_b2, 4).reshape(1, L),
      _bd(hv_w2), jnp.tile(hv_b2, 4).reshape(1, L),
      _bd(w1g), _bd(w1h), _bd(fv_w2), jnp.tile(fv_b2, 4).reshape(1, L),
      _bd(w1f))
    return out.reshape(m, EMB)


def _post_a(sum_a, cnt_a0, cnt_a1, pxa, ga_w2, ga_b2, w1f, fa_w2, fa_b2):
    m = pxa.shape[0]
    bm = 2048
    bm4 = bm // 4
    m4 = m // 4
    grid = _cdiv(m4, bm4)
    L = 4 * EMB

    def body(sa_r, ca_r, px_r, gw2, gb2, w1f_r, fw2, fb2, out_r):
        sarr = sa_r[...]
        s = sarr[0] + sarr[1]
        c = ca_r[...]
        mvals = s / jnp.maximum(c, 1.0)
        gate = jnp.where(c > 0.0, 1.0, 0.0)
        ag = (jnp.dot(mvals, gw2[...], preferred_element_type=F32,
                      precision=lax.Precision.HIGHEST) + gb2[...] * gate)
        hmid = jax.nn.relu(
            px_r[...]
            + jnp.dot(ag, w1f_r[...], preferred_element_type=F32,
                      precision=lax.Precision.HIGHEST))
        out_r[...] = jnp.dot(hmid, fw2[...], preferred_element_type=F32,
                             precision=lax.Precision.HIGHEST) + fb2[...]

    wblk = pl.BlockSpec((L, L), lambda i: (0, 0))
    bblk = pl.BlockSpec((1, L), lambda i: (0, 0))
    cnt_rep = jnp.repeat((cnt_a0 + cnt_a1).reshape(m4, 4), EMB, axis=1)
    out = pl.pallas_call(
        body, grid=(grid,),
        in_specs=[pl.BlockSpec((2, bm4, L), lambda i: (0, i, 0)),
                  pl.BlockSpec((bm4, L), lambda i: (i, 0)),
                  pl.BlockSpec((bm4, L), lambda i: (i, 0)),
                  wblk, bblk, wblk, wblk, bblk],
        out_specs=pl.BlockSpec((bm4, L), lambda i: (i, 0)),
        out_shape=jax.ShapeDtypeStruct((m4, L), F32),
    )(sum_a.reshape(2, m4, L), cnt_rep, pxa.reshape(m4, L),
      _bd(ga_w2), jnp.tile(ga_b2, 4).reshape(1, L), _bd(w1f),
      _bd(fa_w2), jnp.tile(fa_b2, 4).reshape(1, L))
    return out.reshape(m, EMB)


def _pad_idx(idx, e_pad, mod):
    e = idx.shape[0]
    pad = jnp.arange(e_pad - e, dtype=jnp.int32) % mod
    return jnp.concatenate([idx.astype(jnp.int32), pad])


def _pad_idx_off(idx, e_pad, base, mod):
    e = idx.shape[0]
    pad = base + jnp.arange(e_pad - e, dtype=jnp.int32) % mod
    return jnp.concatenate([idx.astype(jnp.int32), pad])


def kernel(x_c, x_v, x_a, edge_index_c2v, edge_index_a2v, edge_attr_c2v,
           edge_attr_a2v, gv_w1, gv_b1, gv_w2, gv_b2, hv_w1, hv_b1, hv_w2,
           hv_b2, fv_w1, fv_b1, fv_w2, fv_b2, ga_w1, ga_b1, ga_w2, ga_b2,
           fa_w1, fa_b1, fa_w2, fa_b2):
    n_v = x_v.shape[0]
    n_a = x_a.shape[0]
    e_c = edge_index_c2v.shape[1]
    e_a = edge_index_a2v.shape[1]

    c2v_s = edge_index_c2v[0]
    c2v_t = edge_index_c2v[1]
    a2v_s = edge_index_a2v[0]
    a2v_t = edge_index_a2v[1]

    zb32 = jnp.zeros((EMB,), F32)

    pv_g, pv_h, pxv_f = _proj(
        x_v, [(gv_w1[:128], zb32), (hv_w1[:128], zb32), (fv_w1[:128], fv_b1)])
    (pc_g,) = _proj(x_c, [(gv_w1[128:256], zb32)])
    pa_h, pa_g, pxa_f = _proj(
        x_a, [(hv_w1[128:256], zb32), (ga_w1[:128], zb32),
              (fa_w1[:128], fa_b1)])

    e_c_pad = _cdiv(e_c, 32 * 112 * 4) * 32 * 112 * 4
    e_a_pad = _cdiv(e_a, 32 * 112 * 4) * 32 * 112 * 4
    (eg,) = _proj_t(jnp.transpose(edge_attr_c2v), [(gv_w1[256:272], gv_b1)],
                    m_out=e_c_pad)
    eh, ea = _proj_t(jnp.transpose(edge_attr_a2v),
                     [(hv_w1[256:272], hv_b1), (ga_w1[160:176], ga_b1)],
                     m_out=e_a_pad)

    n_v_pad = 50176
    n_a_pad = 10240
    c2v_t_g = _pack_perm(_pad_idx(c2v_t, e_c_pad, 256))
    c2v_s_g = _pack_perm(_pad_idx(c2v_s, e_c_pad, 256))
    c2v_t_o = _pack_perm(_pad_idx_off(c2v_t, e_c_pad, n_v, n_v_pad - n_v))
    a2v_t_g = _pack_perm(_pad_idx(a2v_t, e_a_pad, 256))
    a2v_s_g = _pack_perm(_pad_idx(a2v_s, e_a_pad, 256))
    a2v_t_o = _pack_perm(_pad_idx_off(a2v_t, e_a_pad, n_v, n_v_pad - n_v))
    a2v_s_o = _pack_perm(_pad_idx_off(a2v_s, e_a_pad, n_a, n_a_pad - n_a))

    sum_g, cnt_g0, cnt_g1 = _edge_aggr(pv_g, pc_g, eg, c2v_t_g, c2v_s_g,
                                       c2v_t_o, n_v, n_v_pad)
    sum_h, cnt_h0, cnt_h1 = _edge_aggr(pv_h, pa_h, eh, a2v_t_g, a2v_s_g,
                                       a2v_t_o, n_v, n_v_pad)

    pf_g = _post_v(sum_g, cnt_g0, cnt_g1, sum_h, cnt_h0, cnt_h1, pxv_f,
                   gv_w2, gv_b2, hv_w2, hv_b2,
                   fv_w1[128:160], fv_w1[160:192], fv_w2, fv_b2,
                   ga_w1[128:160])

    sum_a, cnt_a0, cnt_a1 = _edge_aggr(pf_g, pa_g, ea, a2v_t_g, a2v_s_g,
                                       a2v_s_o, n_a, n_a_pad)

    return _post_a(sum_a, cnt_a0, cnt_a1, pxa_f, ga_w2, ga_b2,
                   fa_w1[128:160], fa_w2, fa_b2)


# speedup vs baseline: 1.3241x; 1.3241x over previous
import functools

import jax
import jax.numpy as jnp
from jax import lax
from jax.experimental import pallas as pl
from jax.experimental.pallas import tpu as pltpu
from jax.experimental.pallas import tpu_sc as plsc

F32 = jnp.float32
EMB = 32


def _cdiv(a, b):
    return -(-a // b)


def _bd(w):
    return jnp.kron(jnp.eye(4, dtype=F32), w)


def _proj(x, wbs, m_out=None, bm=2048):
    m, k = x.shape
    mo = m_out if m_out is not None else m
    assert m % 4 == 0 and mo % 4 == 0 and bm % 4 == 0
    bm4 = bm // 4
    m4 = m // 4
    mo4 = mo // 4
    n_in_blocks = _cdiv(m4, bm4)
    grid = _cdiv(mo4, bm4)
    nw = len(wbs)
    x4 = x.reshape(m4, 4 * k)

    def body(x_ref, *refs):
        w_refs = refs[:nw]
        b_refs = refs[nw:2 * nw]
        o_refs = refs[2 * nw:]
        xb = x_ref[...]
        for w_r, b_r, o_r in zip(w_refs, b_refs, o_refs):
            o_r[...] = jnp.dot(xb, w_r[...], preferred_element_type=F32,
                               precision=lax.Precision.HIGHEST) + b_r[...]

    in_specs = [pl.BlockSpec((bm4, 4 * k),
                             lambda i, _n=n_in_blocks: (jnp.minimum(i, _n - 1), 0))]
    in_specs += [pl.BlockSpec((4 * k, 4 * EMB), lambda i: (0, 0)) for _ in wbs]
    in_specs += [pl.BlockSpec((1, 4 * EMB), lambda i: (0, 0)) for _ in wbs]
    out_specs = [pl.BlockSpec((bm4, 4 * EMB), lambda i: (i, 0)) for _ in wbs]
    out_shape = [jax.ShapeDtypeStruct((mo4, 4 * EMB), F32) for _ in wbs]
    ws = [_bd(w) for w, _ in wbs]
    bs = [jnp.tile(b, 4).reshape(1, 4 * EMB) for _, b in wbs]
    outs = pl.pallas_call(
        body, grid=(grid,), in_specs=in_specs, out_specs=out_specs,
        out_shape=out_shape)(x4, *ws, *bs)
    return [o.reshape(mo, EMB) for o in outs]


def _edge_aggr(tab_t, tab_s, erows, idx_t, idx_s, idx_o, n_acc, n_acc_pad):
    e_pad = erows.shape[0]
    CH = 112
    NW = 32
    assert e_pad % (NW * CH * 4) == 0
    iters = e_pad // (NW * CH)
    per_w = iters * CH
    assert iters >= 8 and iters % 4 == 0
    zrows = n_acc_pad // 16
    assert zrows % 16 == 0
    zc = next(z for z in range(112, 7, -8) if zrows % z == 0)
    n_zb = zrows // zc
    dr = ((_cdiv(n_acc, 16) + 7) // 8) * 8
    dr_last = n_acc - 15 * dr
    assert dr_last > 0 and dr % 8 == 0 and dr_last % 8 == 0
    dc = ((_cdiv(n_acc, 16) + 127) // 128) * 128
    dc_last = n_acc - 15 * dc
    assert dc_last > 0 and dc % 128 == 0

    mesh = plsc.VectorSubcoreMesh(core_axis_name="c", subcore_axis_name="s")

    @functools.partial(
        pl.kernel, mesh=mesh,
        compiler_params=pltpu.CompilerParams(use_tc_tiling_on_sc=False),
        out_type=[jax.ShapeDtypeStruct((2, n_acc, EMB), F32),
                  jax.ShapeDtypeStruct((n_acc,), F32),
                  jax.ShapeDtypeStruct((n_acc,), F32)],
        scratch_types=(
            [pltpu.VMEM((CH,), jnp.int32)] * 8
            + [pltpu.VMEM((CH, EMB), F32)] * 6
            + [
                pltpu.VMEM((CH,), F32),
                pltpu.VMEM((zc, EMB), F32),
                pltpu.VMEM((zc,), F32),
                pltpu.VMEM_SHARED((n_acc_pad, EMB), F32),
                pltpu.VMEM_SHARED((n_acc_pad,), F32),
                pltpu.SemaphoreType.DMA,
                pltpu.SemaphoreType.DMA,
                pltpu.SemaphoreType.DMA,
                pltpu.SemaphoreType.DMA,
                pltpu.SemaphoreType.DMA,
            ]
        ),
    )
    def k(tab_t_h, tab_s_h, erows_h, idx_t_h, idx_s_h, idx_o_h,
          out_sum, out_cnt0, out_cnt1,
          it0, is0, it1, is1, io0, io1, io2, io3,
          rt0, rs0, re0, rt1, rs1, re1,
          ones_v, zrow_v, zcnt_v,
          sum_sp, cnt_sp, semi0, semi1, semg0, semg1, sems):
        cid = lax.axis_index("c")
        sid = lax.axis_index("s")
        w = cid * 16 + sid

        def zr_body(i, c):
            zrow_v[i, pl.ds(0, 16)] = jnp.zeros((16,), F32)
            zrow_v[i, pl.ds(16, 16)] = jnp.zeros((16,), F32)
            return c
        lax.fori_loop(0, zc, zr_body, 0)

        def zc_body(i, c):
            zcnt_v[pl.ds(i * 16, 16)] = jnp.zeros((16,), F32)
            return c
        lax.fori_loop(0, zc // 16, zc_body, 0)

        def on_body(i, c):
            ones_v[pl.ds(i * 16, 16)] = jnp.ones((16,), F32)
            return c
        lax.fori_loop(0, CH // 16, on_body, 0)

        for q in range(n_zb):
            pltpu.sync_copy(zrow_v,
                            sum_sp.at[pl.ds(sid * zrows + q * zc, zc)])
            pltpu.sync_copy(zcnt_v,
                            cnt_sp.at[pl.ds(sid * zrows + q * zc, zc)])
        plsc.subcore_barrier()

        idxg = ((it0, is0), (it1, is1))
        idxo = (io0, io1, io2, io3)
        datb = ((rt0, rs0, re0), (rt1, rs1, re1))
        semi = (semi0, semi1)
        semg = (semg0, semg1)

        def issue_idx(i, b, o):
            base = pl.multiple_of(w * per_w + i * CH, 8)
            pltpu.async_copy(idx_t_h.at[pl.ds(base, CH)], idxg[b][0], semi[b])
            pltpu.async_copy(idx_s_h.at[pl.ds(base, CH)], idxg[b][1], semi[b])
            pltpu.async_copy(idx_o_h.at[pl.ds(base, CH)], idxo[o], semi[b])

        def wait_idx(b, o):
            for dst in (idxg[b][0], idxg[b][1], idxo[o]):
                pltpu.make_async_copy(idx_t_h.at[pl.ds(0, CH)], dst,
                                      semi[b]).wait()

        def issue_gather(i, b):
            base = pl.multiple_of(w * per_w + i * CH, 8)
            pltpu.async_copy(erows_h.at[pl.ds(base, CH)], datb[b][2], semg[b])
            pltpu.async_copy(tab_t_h.at[idxg[b][0]], datb[b][0], semg[b])
            pltpu.async_copy(tab_s_h.at[idxg[b][1]], datb[b][1], semg[b])

        def wait_gather(b):
            for dst in datb[b]:
                pltpu.make_async_copy(erows_h.at[pl.ds(0, CH)], dst,
                                      semg[b]).wait()

        def drain_scatter():
            pltpu.make_async_copy(erows_h.at[pl.ds(0, CH)], re0, sems).wait()

        def half(i, b, o):
            rt, rs, re = datb[b]
            wait_gather(b)

            def cbody(blk, cc):
                r0 = blk * 16
                for rr in range(16):
                    r = r0 + rr
                    for col in (0, 16):
                        z = (rt[r, pl.ds(col, 16)] + rs[r, pl.ds(col, 16)]
                             + re[r, pl.ds(col, 16)])
                        re[r, pl.ds(col, 16)] = jnp.maximum(z, 0.0)
                return cc
            lax.fori_loop(0, CH // 16, cbody, 0)

            @pl.when(i >= 1)
            def _():
                drain_scatter()
            pltpu.async_copy(re, sum_sp.at[idxo[o]], sems, add=True)
            pltpu.sync_copy(ones_v, cnt_sp.at[idxo[o]], add=True)

            @pl.when(i + 2 < iters)
            def _():
                issue_idx(i + 2, b, (o + 2) % 4)

            @pl.when(i + 1 < iters)
            def _():
                wait_idx(b ^ 1, (o + 1) % 4)
                issue_gather(i + 1, b ^ 1)

        issue_idx(0, 0, 0)
        wait_idx(0, 0)
        issue_gather(0, 0)
        issue_idx(1, 1, 1)

        def it_body(q, c):
            for j in range(4):
                half(4 * q + j, j & 1, j)
            return c
        lax.fori_loop(0, iters // 4, it_body, 0)
        drain_scatter()

        plsc.subcore_barrier()

        @pl.when(sid < 15)
        def _():
            pltpu.sync_copy(sum_sp.at[pl.ds(sid * dr, dr)],
                            out_sum.at[cid, pl.ds(sid * dr, dr)])

        @pl.when(sid == 15)
        def _():
            pltpu.sync_copy(sum_sp.at[pl.ds(15 * dr, dr_last)],
                            out_sum.at[cid, pl.ds(15 * dr, dr_last)])

        for c, out_cnt in ((0, out_cnt0), (1, out_cnt1)):
            @pl.when(jnp.logical_and(cid == c, sid < 15))
            def _(out_cnt=out_cnt):
                pltpu.sync_copy(cnt_sp.at[pl.ds(sid * dc, dc)],
                                out_cnt.at[pl.ds(sid * dc, dc)])

            @pl.when(jnp.logical_and(cid == c, sid == 15))
            def _(out_cnt=out_cnt):
                pltpu.sync_copy(cnt_sp.at[pl.ds(15 * dc, dc_last)],
                                out_cnt.at[pl.ds(15 * dc, dc_last)])

    return k(tab_t, tab_s, erows, idx_t, idx_s, idx_o)


def _post_v(sum_g, cnt_g0, cnt_g1, sum_h, cnt_h0, cnt_h1, pxv,
            gv_w2, gv_b2, hv_w2, hv_b2, w1g, w1h, fv_w2, fv_b2, w1f):
    m = cnt_g0.shape[0]
    bm = 2048
    bm4 = bm // 4
    m4 = m // 4
    grid = _cdiv(m4, bm4)
    L = 4 * EMB

    def rep(cnt):
        return jnp.repeat(cnt.reshape(m4, 4), EMB, axis=1)

    def body(sg_r, cg_r, sh_r, ch_r, px_r, gw2, gb2, hw2, hb2,
             w1g_r, w1h_r, fw2, fb2, w1f_r, out_r):
        def aggr(s_r, c_r, w2, b2):
            sarr = s_r[...]
            s = sarr[0] + sarr[1]
            c = c_r[...]
            mvals = s / jnp.maximum(c, 1.0)
            gate = jnp.where(c > 0.0, 1.0, 0.0)
            return (jnp.dot(mvals, w2[...], preferred_element_type=F32,
                            precision=lax.Precision.HIGHEST)
                    + b2[...] * gate)
        ag = aggr(sg_r, cg_r, gw2, gb2)
        ah = aggr(sh_r, ch_r, hw2, hb2)
        hmid = jax.nn.relu(
            px_r[...]
            + jnp.dot(ag, w1g_r[...], preferred_element_type=F32,
                      precision=lax.Precision.HIGHEST)
            + jnp.dot(ah, w1h_r[...], preferred_element_type=F32,
                      precision=lax.Precision.HIGHEST))
        fv = jnp.dot(hmid, fw2[...], preferred_element_type=F32,
                     precision=lax.Precision.HIGHEST) + fb2[...]
        out_r[...] = jnp.dot(fv, w1f_r[...], preferred_element_type=F32,
                             precision=lax.Precision.HIGHEST)

    sblk = pl.BlockSpec((2, bm4, L), lambda i: (0, i, 0))
    cblk = pl.BlockSpec((bm4, L), lambda i: (i, 0))
    wblk = pl.BlockSpec((L, L), lambda i: (0, 0))
    bblk = pl.BlockSpec((1, L), lambda i: (0, 0))
    out = pl.pallas_call(
        body, grid=(grid,),
        in_specs=[sblk, cblk, sblk, cblk, cblk,
                  wblk, bblk, wblk, bblk, wblk, wblk, wblk, bblk, wblk],
        out_specs=pl.BlockSpec((bm4, L), lambda i: (i, 0)),
        out_shape=jax.ShapeDtypeStruct((m4, L), F32),
    )(sum_g.reshape(2, m4, L), rep(cnt_g0 + cnt_g1),
      sum_h.reshape(2, m4, L), rep(cnt_h0 + cnt_h1),
      pxv.reshape(m4, L),
      _bd(gv_w2), jnp.tile(gv_b2, 4).reshape(1, L),
      _bd(hv_w2), jnp.tile(hv_b2, 4).reshape(1, L),
      _bd(w1g), _bd(w1h), _bd(fv_w2), jnp.tile(fv_b2, 4).reshape(1, L),
      _bd(w1f))
    return out.reshape(m, EMB)


def _post_a(sum_a, cnt_a0, cnt_a1, pxa, ga_w2, ga_b2, w1f, fa_w2, fa_b2):
    m = pxa.shape[0]
    bm = 2048
    bm4 = bm // 4
    m4 = m // 4
    grid = _cdiv(m4, bm4)
    L = 4 * EMB

    def body(sa_r, ca_r, px_r, gw2, gb2, w1f_r, fw2, fb2, out_r):
        sarr = sa_r[...]
        s = sarr[0] + sarr[1]
        c = ca_r[...]
        mvals = s / jnp.maximum(c, 1.0)
        gate = jnp.where(c > 0.0, 1.0, 0.0)
        ag = (jnp.dot(mvals, gw2[...], preferred_element_type=F32,
                      precision=lax.Precision.HIGHEST) + gb2[...] * gate)
        hmid = jax.nn.relu(
            px_r[...]
            + jnp.dot(ag, w1f_r[...], preferred_element_type=F32,
                      precision=lax.Precision.HIGHEST))
        out_r[...] = jnp.dot(hmid, fw2[...], preferred_element_type=F32,
                             precision=lax.Precision.HIGHEST) + fb2[...]

    wblk = pl.BlockSpec((L, L), lambda i: (0, 0))
    bblk = pl.BlockSpec((1, L), lambda i: (0, 0))
    cnt_rep = jnp.repeat((cnt_a0 + cnt_a1).reshape(m4, 4), EMB, axis=1)
    out = pl.pallas_call(
        body, grid=(grid,),
        in_specs=[pl.BlockSpec((2, bm4, L), lambda i: (0, i, 0)),
                  pl.BlockSpec((bm4, L), lambda i: (i, 0)),
                  pl.BlockSpec((bm4, L), lambda i: (i, 0)),
                  wblk, bblk, wblk, wblk, bblk],
        out_specs=pl.BlockSpec((bm4, L), lambda i: (i, 0)),
        out_shape=jax.ShapeDtypeStruct((m4, L), F32),
    )(sum_a.reshape(2, m4, L), cnt_rep, pxa.reshape(m4, L),
      _bd(ga_w2), jnp.tile(ga_b2, 4).reshape(1, L), _bd(w1f),
      _bd(fa_w2), jnp.tile(fa_b2, 4).reshape(1, L))
    return out.reshape(m, EMB)


def _pad_idx(idx, e_pad, mod):
    e = idx.shape[0]
    pad = jnp.arange(e_pad - e, dtype=jnp.int32) % mod
    return jnp.concatenate([idx.astype(jnp.int32), pad])


def _pad_idx_off(idx, e_pad, base, mod):
    e = idx.shape[0]
    pad = base + jnp.arange(e_pad - e, dtype=jnp.int32) % mod
    return jnp.concatenate([idx.astype(jnp.int32), pad])


def kernel(x_c, x_v, x_a, edge_index_c2v, edge_index_a2v, edge_attr_c2v,
           edge_attr_a2v, gv_w1, gv_b1, gv_w2, gv_b2, hv_w1, hv_b1, hv_w2,
           hv_b2, fv_w1, fv_b1, fv_w2, fv_b2, ga_w1, ga_b1, ga_w2, ga_b2,
           fa_w1, fa_b1, fa_w2, fa_b2):
    n_v = x_v.shape[0]
    n_a = x_a.shape[0]
    e_c = edge_index_c2v.shape[1]
    e_a = edge_index_a2v.shape[1]

    c2v_s = edge_index_c2v[0]
    c2v_t = edge_index_c2v[1]
    a2v_s = edge_index_a2v[0]
    a2v_t = edge_index_a2v[1]

    zb32 = jnp.zeros((EMB,), F32)

    pv_g, pv_h, pxv_f = _proj(
        x_v, [(gv_w1[:128], zb32), (hv_w1[:128], zb32), (fv_w1[:128], fv_b1)])
    (pc_g,) = _proj(x_c, [(gv_w1[128:256], zb32)])
    pa_h, pa_g, pxa_f = _proj(
        x_a, [(hv_w1[128:256], zb32), (ga_w1[:128], zb32),
              (fa_w1[:128], fa_b1)])

    e_c_pad = _cdiv(e_c, 32 * 112 * 4) * 32 * 112 * 4
    e_a_pad = _cdiv(e_a, 32 * 112 * 4) * 32 * 112 * 4
    (eg,) = _proj(edge_attr_c2v, [(gv_w1[256:272], gv_b1)], m_out=e_c_pad,
                  bm=4096)
    eh, ea = _proj(edge_attr_a2v, [(hv_w1[256:272], hv_b1),
                                   (ga_w1[160:176], ga_b1)], m_out=e_a_pad,
                   bm=4096)

    n_v_pad = 50176
    n_a_pad = 10240
    c2v_t_g = _pad_idx(c2v_t, e_c_pad, 256)
    c2v_s_g = _pad_idx(c2v_s, e_c_pad, 256)
    c2v_t_o = _pad_idx_off(c2v_t, e_c_pad, n_v, n_v_pad - n_v)
    a2v_t_g = _pad_idx(a2v_t, e_a_pad, 256)
    a2v_s_g = _pad_idx(a2v_s, e_a_pad, 256)
    a2v_t_o = _pad_idx_off(a2v_t, e_a_pad, n_v, n_v_pad - n_v)
    a2v_s_o = _pad_idx_off(a2v_s, e_a_pad, n_a, n_a_pad - n_a)

    sum_g, cnt_g0, cnt_g1 = _edge_aggr(pv_g, pc_g, eg, c2v_t_g, c2v_s_g,
                                       c2v_t_o, n_v, n_v_pad)
    sum_h, cnt_h0, cnt_h1 = _edge_aggr(pv_h, pa_h, eh, a2v_t_g, a2v_s_g,
                                       a2v_t_o, n_v, n_v_pad)

    pf_g = _post_v(sum_g, cnt_g0, cnt_g1, sum_h, cnt_h0, cnt_h1, pxv_f,
                   gv_w2, gv_b2, hv_w2, hv_b2,
                   fv_w1[128:160], fv_w1[160:192], fv_w2, fv_b2,
                   ga_w1[128:160])

    sum_a, cnt_a0, cnt_a1 = _edge_aggr(pf_g, pa_g, ea, a2v_t_g, a2v_s_g,
                                       a2v_s_o, n_a, n_a_pad)

    return _post_a(sum_a, cnt_a0, cnt_a1, pxa_f, ga_w2, ga_b2,
                   fa_w1[128:160], fa_w2, fa_b2)
